# Initial kernel scaffold; baseline (speedup 1.0000x reference)
import jax
import jax.numpy as jnp
from jax.experimental import pallas as pl


def kernel(batch, embedding_table):
    raise NotImplementedError("write your pallas kernel here")


# speedup vs baseline: 2.9172x; 2.9172x over previous
import functools

import numpy as np
import jax
import jax.numpy as jnp
from jax import lax
from jax.experimental import pallas as pl
from jax.experimental.pallas import tpu as pltpu
from jax.experimental.pallas import tpu_sc as plsc

VOCAB = 63
D = 16
SEQ = 19
BATCH = 16384
TOKENS = BATCH * SEQ

NC = 2
NS = 16
NW = NC * NS
PER_W = TOKENS // NW
GROUP = 128
GROUPS = PER_W // GROUP
CHUNK_GROUPS = 19
CHUNKS = GROUPS // CHUNK_GROUPS
CHUNK_ROWS = CHUNK_GROUPS * GROUP


def _pe_np() -> np.ndarray:
    even_i = np.arange(0, D, 2, dtype=np.float32)
    denom = np.power(np.float32(10000.0), even_i / np.float32(D))
    pos = np.arange(SEQ, dtype=np.float32).reshape(SEQ, 1)
    stacked = np.stack([np.sin(pos / denom), np.cos(pos / denom)], axis=-1)
    return stacked.reshape(SEQ, D).astype(np.float32)


_PE = _pe_np()

_POSOFF = (VOCAB * (np.arange(CHUNK_GROUPS * GROUP, dtype=np.int32) % SEQ)
           ).reshape(CHUNK_GROUPS, GROUP)


def _fuse_body(tab_ref, pe_ref, out_ref):
    out_ref[...] = tab_ref[...][None, :, :] + pe_ref[...][:, None, :]


_fuse = pl.pallas_call(
    _fuse_body,
    out_shape=jax.ShapeDtypeStruct((SEQ, VOCAB, D), jnp.float32),
)


def _sc_body(idx_hbm, posoff_hbm, fused_hbm, out_hbm, idx_v, po_v, rows_v,
             gsem, ssem):
    wid = lax.axis_index("s") * NC + lax.axis_index("c")

    pltpu.sync_copy(idx_hbm.at[wid], idx_v)
    pltpu.sync_copy(posoff_hbm, po_v)

    def _xform(g, _):
        po_row = lax.rem(g, CHUNK_GROUPS)
        for j in range(GROUP // 16):
            sl = pl.ds(j * 16, 16)
            idx_v[g, sl] = idx_v[g, sl] + po_v[po_row, sl]
        return _
    lax.fori_loop(0, GROUPS, _xform, None)

    for c in range(CHUNKS):
        copies = []
        for g in range(CHUNK_GROUPS):
            gg = c * CHUNK_GROUPS + g
            copies.append(pltpu.async_copy(
                fused_hbm.at[idx_v.at[gg]],
                rows_v.at[pl.ds(g * GROUP, GROUP)],
                gsem))
        for cp in copies:
            cp.wait()
        pltpu.sync_copy(
            rows_v,
            out_hbm.at[pl.ds(wid * PER_W + c * CHUNK_ROWS, CHUNK_ROWS)])


_sc_call = functools.partial(
    pl.kernel,
    out_type=jax.ShapeDtypeStruct((TOKENS, D), jnp.float32),
    mesh=plsc.VectorSubcoreMesh(core_axis_name="c", subcore_axis_name="s"),
    compiler_params=pltpu.CompilerParams(use_tc_tiling_on_sc=False),
    scratch_types=[
        pltpu.VMEM((NW * GROUPS // NW, GROUP), jnp.int32),
        pltpu.VMEM((CHUNK_GROUPS, GROUP), jnp.int32),
        pltpu.VMEM((CHUNK_ROWS, D), jnp.float32),
        pltpu.SemaphoreType.DMA,
        pltpu.SemaphoreType.DMA,
    ],
)(_sc_body)


def kernel(batch, embedding_table):
    fused = _fuse(embedding_table.astype(jnp.float32), jnp.asarray(_PE))
    fused = fused.reshape(SEQ * VOCAB, D)
    idx = batch.astype(jnp.int32).reshape(NW, GROUPS, GROUP)
    out = _sc_call(idx, jnp.asarray(_POSOFF), fused)
    return out.reshape(BATCH, SEQ, D)


# speedup vs baseline: 16.8247x; 5.7674x over previous
import functools

import numpy as np
import jax
import jax.numpy as jnp
from jax import lax
from jax.experimental import pallas as pl
from jax.experimental.pallas import tpu as pltpu
from jax.experimental.pallas import tpu_sc as plsc

VOCAB = 63
VOCAB_PAD = 64
D = 16
SEQ = 19
BATCH = 16384
TOKENS = BATCH * SEQ

NC = 2
NS = 16
NW = NC * NS
BBLK = BATCH // 128
UNITS = SEQ * BBLK
UNITS_W = UNITS // NW
PER_W = UNITS_W * 128


def _pe_np() -> np.ndarray:
    even_i = np.arange(0, D, 2, dtype=np.float32)
    denom = np.power(np.float32(10000.0), even_i / np.float32(D))
    pos = np.arange(SEQ, dtype=np.float32).reshape(SEQ, 1)
    stacked = np.stack([np.sin(pos / denom), np.cos(pos / denom)], axis=-1)
    return stacked.reshape(SEQ, D).astype(np.float32)


_PE = _pe_np()


def _fuse_body(tab_ref, pe_ref, out_ref):
    tab_t = jnp.transpose(tab_ref[...], (1, 0))
    pe_t = jnp.transpose(pe_ref[...], (1, 0))
    out_ref[...] = pe_t[:, :, None] + tab_t[:, None, :]


_fuse = pl.pallas_call(
    _fuse_body,
    out_shape=jax.ShapeDtypeStruct((D, SEQ, VOCAB_PAD), jnp.float32),
)


def _sc_body(idx_hbm, fusedt_hbm, out_hbm, idx_v, tab_v, tile_v, ssem):
    wid = lax.axis_index("s") * NC + lax.axis_index("c")

    pltpu.sync_copy(idx_hbm.at[pl.ds(wid * PER_W, PER_W)], idx_v)
    pltpu.sync_copy(fusedt_hbm, tab_v)

    dvecs = [jnp.full((16,), d, jnp.int32) for d in range(D)]

    def _unit(i, _):
        u = wid * UNITS_W + i
        p = lax.div(u, BBLK)
        bblk = lax.rem(u, BBLK)
        poff = p * VOCAB_PAD
        for g in range(8):
            fidx = idx_v[pl.ds(i * 128 + g * 16, 16)] + poff
            for d in range(D):
                tile_v[d, pl.ds(g * 16, 16)] = plsc.load_gather(
                    tab_v, [dvecs[d], fidx])
        pltpu.sync_copy(tile_v.at[pl.ds(0, 8)], out_hbm.at[p, 0, bblk])
        pltpu.sync_copy(tile_v.at[pl.ds(8, 8)], out_hbm.at[p, 1, bblk])
        return _

    lax.fori_loop(0, UNITS_W, _unit, None)


_sc_call = functools.partial(
    pl.kernel,
    out_type=jax.ShapeDtypeStruct((SEQ, 2, BBLK, 8, 128), jnp.float32),
    mesh=plsc.VectorSubcoreMesh(core_axis_name="c", subcore_axis_name="s"),
    compiler_params=pltpu.CompilerParams(use_tc_tiling_on_sc=False,
                                         needs_layout_passes=False),
    scratch_types=[
        pltpu.VMEM((PER_W,), jnp.int32),
        pltpu.VMEM((D, SEQ * VOCAB_PAD), jnp.float32),
        pltpu.VMEM((D, 128), jnp.float32),
        pltpu.SemaphoreType.DMA,
    ],
)(_sc_body)


def kernel(batch, embedding_table):
    tab = jnp.pad(embedding_table.astype(jnp.float32),
                  ((0, VOCAB_PAD - VOCAB), (0, 0)))
    fusedt = _fuse(tab, jnp.asarray(_PE))
    fusedt = fusedt.reshape(D, SEQ * VOCAB_PAD)
    idx = batch.astype(jnp.int32).T.reshape(TOKENS)
    out5 = _sc_call(idx, fusedt)
    out = out5.transpose(2, 4, 0, 1, 3).reshape(BATCH, SEQ, D)
    return out
